# baseline (speedup 1.0000x reference)
import jax
import jax.numpy as jnp
from jax.experimental import pallas as pl
from jax.experimental.pallas import tpu as pltpu


_WIDE_ROWS = 8
_BLOCK_COLS = 131072


def _copy_body(u_ref, i_ref, out_u_ref, out_i_ref):
    out_u_ref[...] = u_ref[...]
    out_i_ref[...] = i_ref[...]


def kernel(user_history, user_table, item_table):
    del user_history
    n_rows, dim = user_table.shape
    wide_cols = (n_rows * dim) // _WIDE_ROWS
    u = user_table.reshape(_WIDE_ROWS, wide_cols)
    i = item_table.reshape(_WIDE_ROWS, wide_cols)
    grid = (wide_cols // _BLOCK_COLS,)
    spec = pl.BlockSpec((_WIDE_ROWS, _BLOCK_COLS), lambda j: (0, j))
    out_shapes = (
        jax.ShapeDtypeStruct((_WIDE_ROWS, wide_cols), user_table.dtype),
        jax.ShapeDtypeStruct((_WIDE_ROWS, wide_cols), item_table.dtype),
    )
    user_emb, item_emb = pl.pallas_call(
        _copy_body,
        grid=grid,
        out_shape=out_shapes,
        in_specs=[spec, spec],
        out_specs=(spec, spec),
    )(u, i)
    return (user_emb.reshape(n_rows, dim), item_emb.reshape(n_rows, dim))


# speedup vs baseline: 9.5898x; 9.5898x over previous
import jax
import jax.numpy as jnp
from jax.experimental import pallas as pl
from jax.experimental.pallas import tpu as pltpu


_WIDE_COLS = 128
_BLOCK_ROWS = 10000


def _copy_body(u_ref, i_ref, out_u_ref, out_i_ref):
    out_u_ref[...] = u_ref[...]
    out_i_ref[...] = i_ref[...]


def kernel(user_history, user_table, item_table):
    del user_history
    n_rows, dim = user_table.shape
    wide_rows = (n_rows * dim) // _WIDE_COLS
    u = user_table.reshape(wide_rows, _WIDE_COLS)
    i = item_table.reshape(wide_rows, _WIDE_COLS)
    grid = (wide_rows // _BLOCK_ROWS,)
    spec = pl.BlockSpec((_BLOCK_ROWS, _WIDE_COLS), lambda j: (j, 0))
    out_shapes = (
        jax.ShapeDtypeStruct((wide_rows, _WIDE_COLS), user_table.dtype),
        jax.ShapeDtypeStruct((wide_rows, _WIDE_COLS), item_table.dtype),
    )
    user_emb, item_emb = pl.pallas_call(
        _copy_body,
        grid=grid,
        out_shape=out_shapes,
        in_specs=[spec, spec],
        out_specs=(spec, spec),
        compiler_params=pltpu.CompilerParams(
            dimension_semantics=("arbitrary",),
        ),
    )(u, i)
    return (user_emb.reshape(n_rows, dim), item_emb.reshape(n_rows, dim))


# speedup vs baseline: 12.0620x; 1.2578x over previous
import jax
import jax.numpy as jnp
from jax.experimental import pallas as pl
from jax.experimental.pallas import tpu as pltpu

_CHUNK_ROWS = 10000
_N_BUF = 8


def _copy_body(u_ref, i_ref, out_u_ref, out_i_ref, bufs, in_sems, out_sems):
    n_rows = u_ref.shape[0]
    n_chunks = n_rows // _CHUNK_ROWS
    tasks = []
    for k in range(n_chunks):
        tasks.append((u_ref, out_u_ref, k))
        tasks.append((i_ref, out_i_ref, k))

    def start_in(t):
        src, _, k = tasks[t]
        slot = t % _N_BUF
        pltpu.make_async_copy(
            src.at[pl.ds(k * _CHUNK_ROWS, _CHUNK_ROWS), :],
            bufs.at[slot],
            in_sems.at[slot],
        ).start()

    def wait_in(t):
        src, _, k = tasks[t]
        slot = t % _N_BUF
        pltpu.make_async_copy(
            src.at[pl.ds(k * _CHUNK_ROWS, _CHUNK_ROWS), :],
            bufs.at[slot],
            in_sems.at[slot],
        ).wait()

    def start_out(t):
        _, dst, k = tasks[t]
        slot = t % _N_BUF
        pltpu.make_async_copy(
            bufs.at[slot],
            dst.at[pl.ds(k * _CHUNK_ROWS, _CHUNK_ROWS), :],
            out_sems.at[slot],
        ).start()

    def wait_out(t):
        _, dst, k = tasks[t]
        slot = t % _N_BUF
        pltpu.make_async_copy(
            bufs.at[slot],
            dst.at[pl.ds(k * _CHUNK_ROWS, _CHUNK_ROWS), :],
            out_sems.at[slot],
        ).wait()

    T = len(tasks)
    for t in range(min(_N_BUF, T)):
        start_in(t)
    for t in range(T):
        wait_in(t)
        start_out(t)
        nt = t + _N_BUF
        if nt < T:
            wait_out(t)
            start_in(nt)
    for t in range(max(T - _N_BUF, 0), T):
        wait_out(t)


def kernel(user_history, user_table, item_table):
    del user_history
    n_rows, dim = user_table.shape
    out_shapes = (
        jax.ShapeDtypeStruct(user_table.shape, user_table.dtype),
        jax.ShapeDtypeStruct(item_table.shape, item_table.dtype),
    )
    user_emb, item_emb = pl.pallas_call(
        _copy_body,
        out_shape=out_shapes,
        in_specs=[
            pl.BlockSpec(memory_space=pl.ANY),
            pl.BlockSpec(memory_space=pl.ANY),
        ],
        out_specs=(
            pl.BlockSpec(memory_space=pl.ANY),
            pl.BlockSpec(memory_space=pl.ANY),
        ),
        scratch_shapes=[
            pltpu.VMEM((_N_BUF, _CHUNK_ROWS, dim), jnp.float32),
            pltpu.SemaphoreType.DMA((_N_BUF,)),
            pltpu.SemaphoreType.DMA((_N_BUF,)),
        ],
        compiler_params=pltpu.CompilerParams(
            vmem_limit_bytes=110 * 1024 * 1024,
        ),
    )(user_table, item_table)
    return (user_emb, item_emb)
